# baseline (speedup 1.0000x reference)
import functools

import jax
import jax.numpy as jnp
from jax import lax
from jax.experimental import pallas as pl
from jax.experimental.pallas import tpu as pltpu
from jax.experimental.pallas import tpu_sc as plsc

EMB_SIZE_C = 1000000
D = 64
B = 4096
NNEG = 5
LAMB_C = 0.1

NC = 2
NS = 16
NW = NC * NS
BPW = B // NW
TPW = B * NNEG // NW


def _sc_gather_body(pos_u_hbm, pos_v_hbm, negf_hbm, u_emb_hbm, v_emb_hbm,
                    massf_hbm,
                    pd_hbm, pd2_hbm, a_hbm, nm_hbm,
                    idxu_v, idxv_v, idxn_v, eu_v, ev_v, en_v,
                    mu_v, mv_v, mn_v, pd_v, pd2_v, a_v, nm_v, sem):
    wid = lax.axis_index("s") * NC + lax.axis_index("c")
    base = wid * BPW
    nbase = wid * TPW

    pltpu.sync_copy(pos_u_hbm.at[pl.ds(base, BPW)], idxu_v)
    pltpu.sync_copy(pos_v_hbm.at[pl.ds(base, BPW)], idxv_v)
    pltpu.sync_copy(negf_hbm.at[pl.ds(nbase, TPW)], idxn_v)

    pltpu.async_copy(u_emb_hbm.at[idxu_v], eu_v, sem).wait()
    pltpu.async_copy(v_emb_hbm.at[idxv_v], ev_v, sem).wait()
    pltpu.async_copy(v_emb_hbm.at[idxn_v], en_v, sem).wait()
    pltpu.async_copy(massf_hbm.at[idxu_v], mu_v, sem).wait()
    pltpu.async_copy(massf_hbm.at[idxv_v], mv_v, sem).wait()
    pltpu.async_copy(massf_hbm.at[idxn_v], mn_v, sem).wait()

    for g in range(BPW // 16):
        sl = pl.ds(16 * g, 16)
        a_v[sl] = mu_v[sl] * mv_v[sl]

    lane = lax.iota(jnp.int32, 16)
    for g in range(TPW // 16):
        sl = pl.ds(16 * g, 16)
        src = lax.shift_right_logical((lane + 16 * g) * 52429, 18)
        nm_v[sl] = plsc.load_gather(mu_v, [src]) * mn_v[sl]

    def pos_row(r, carry):
        acc = jnp.zeros((16,), jnp.float32)
        for c in range(D // 16):
            sl = pl.ds(16 * c, 16)
            du = eu_v[r, sl] - ev_v[r, sl]
            acc = acc + du * du
        pd_v[r, :] = acc
        return carry

    lax.fori_loop(0, BPW, pos_row, 0)

    def neg_row(t, carry):
        r = t // NNEG
        acc = jnp.zeros((16,), jnp.float32)
        for c in range(D // 16):
            sl = pl.ds(16 * c, 16)
            du = eu_v[r, sl] - en_v[t, sl]
            acc = acc + du * du
        pd2_v[t, :] = acc
        return carry

    lax.fori_loop(0, TPW, neg_row, 0)

    pltpu.sync_copy(pd_v, pd_hbm.at[pl.ds(base, BPW)])
    pltpu.sync_copy(pd2_v, pd2_hbm.at[pl.ds(nbase, TPW)])
    pltpu.sync_copy(a_v, a_hbm.at[pl.ds(base, BPW)])
    pltpu.sync_copy(nm_v, nm_hbm.at[pl.ds(nbase, TPW)])


@functools.lru_cache(maxsize=1)
def _make_sc_gather():
    return functools.partial(
        pl.kernel,
        out_type=[
        jax.ShapeDtypeStruct((B, 16), jnp.float32),
        jax.ShapeDtypeStruct((B * NNEG, 16), jnp.float32),
        jax.ShapeDtypeStruct((B,), jnp.float32),
        jax.ShapeDtypeStruct((B * NNEG,), jnp.float32),
        ],
        mesh=plsc.VectorSubcoreMesh(core_axis_name="c", subcore_axis_name="s"),
        compiler_params=pltpu.CompilerParams(
            use_tc_tiling_on_sc=False, needs_layout_passes=False),
        scratch_types=[
            pltpu.VMEM((BPW,), jnp.int32),
            pltpu.VMEM((BPW,), jnp.int32),
            pltpu.VMEM((TPW,), jnp.int32),
            pltpu.VMEM((BPW, D), jnp.float32),
            pltpu.VMEM((BPW, D), jnp.float32),
            pltpu.VMEM((TPW, D), jnp.float32),
            pltpu.VMEM((BPW,), jnp.float32),
            pltpu.VMEM((BPW,), jnp.float32),
            pltpu.VMEM((TPW,), jnp.float32),
            pltpu.VMEM((BPW, 16), jnp.float32),
            pltpu.VMEM((TPW, 16), jnp.float32),
            pltpu.VMEM((BPW,), jnp.float32),
            pltpu.VMEM((TPW,), jnp.float32),
            pltpu.SemaphoreType.DMA,
        ],
    )(_sc_gather_body)


def _softplus(x):
    return jnp.maximum(x, 0.0) + jnp.log1p(jnp.exp(-jnp.abs(x)))


def _tc_score_body(pd_ref, pd2_ref, a_ref, nm_ref, out_ref):
    arow = a_ref[...]

    def blk(i, acc):
        pblk = pd_ref[pl.ds(i * 128, 128), :]
        dist = jnp.sum(pblk, axis=1, keepdims=True)
        bblk = LAMB_C * jnp.log(dist)
        x = jnp.clip(arow - bblk, -10.0, 10.0)
        return acc + jnp.sum(_softplus(-x))

    s1 = lax.fori_loop(0, B // 128, blk, jnp.float32(0.0))

    d2 = jnp.sum(pd2_ref[...], axis=1)
    q = jnp.clip(nm_ref[...] - LAMB_C * jnp.log(d2), -10.0, 10.0)
    s2 = jnp.sum(_softplus(q))

    out_ref[0, 0] = s1 / (B * B) + s2 / B


def kernel(pos_u, pos_v, neg_v, u_emb, v_emb, mass_tbl):
    pos_u = pos_u.astype(jnp.int32)
    pos_v = pos_v.astype(jnp.int32)
    negf = neg_v.astype(jnp.int32).reshape(B * NNEG)
    massf = mass_tbl.reshape(EMB_SIZE_C)

    pd, pd2, av, nmv = _make_sc_gather()(pos_u, pos_v, negf, u_emb, v_emb, massf)

    out = pl.pallas_call(
        _tc_score_body,
        out_shape=jax.ShapeDtypeStruct((1, 1), jnp.float32),
        out_specs=pl.BlockSpec(memory_space=pltpu.SMEM),
    )(pd, pd2, av.reshape(1, B), nmv)
    return out.reshape(())
